# baseline (speedup 1.0000x reference)
import jax
import jax.numpy as jnp
from jax.experimental import pallas as pl
from jax.experimental.pallas import tpu as pltpu

_BM = 80
_NSPLIT = 5
_ROWS = _BM * _NSPLIT


def _gcn_body(*refs):
    adj_refs = refs[:_NSPLIT]
    x_ref, w_ref, b_ref, out_ref = refs[_NSPLIT:]
    for s, adj_ref in enumerate(adj_refs):
        part = jnp.dot(adj_ref[...], x_ref[...], preferred_element_type=jnp.float32)
        out_ref[s * _BM : (s + 1) * _BM, :] = (
            jnp.dot(part, w_ref[...], preferred_element_type=jnp.float32) + b_ref[...]
        )


def kernel(x, adj, W, b):
    n, din = x.shape
    dout = W.shape[1]
    b2 = b.reshape(1, dout)
    adj_specs = [
        pl.BlockSpec((_BM, n), lambda i, s=s: (i * _NSPLIT + s, 0))
        for s in range(_NSPLIT)
    ]
    return pl.pallas_call(
        _gcn_body,
        grid=(pl.cdiv(n, _ROWS),),
        in_specs=adj_specs
        + [
            pl.BlockSpec((n, din), lambda i: (0, 0)),
            pl.BlockSpec((din, dout), lambda i: (0, 0)),
            pl.BlockSpec((1, dout), lambda i: (0, 0)),
        ],
        out_specs=pl.BlockSpec((_ROWS, dout), lambda i: (i, 0)),
        out_shape=jax.ShapeDtypeStruct((n, dout), jnp.float32),
        compiler_params=pltpu.CompilerParams(
            dimension_semantics=("parallel",),
        ),
    )(*([adj] * _NSPLIT), x, W, b2)


# speedup vs baseline: 1.0245x; 1.0245x over previous
import jax
import jax.numpy as jnp
from jax.experimental import pallas as pl
from jax.experimental.pallas import tpu as pltpu

_BM = 400


def _gcn_body(adj_ref, x_ref, w_ref, b_ref, out_ref):
    ax = jnp.dot(adj_ref[...], x_ref[...], preferred_element_type=jnp.float32)
    out_ref[...] = (
        jnp.dot(ax, w_ref[...], preferred_element_type=jnp.float32) + b_ref[...]
    )


def kernel(x, adj, W, b):
    n, din = x.shape
    dout = W.shape[1]
    b2 = b.reshape(1, dout)
    return pl.pallas_call(
        _gcn_body,
        grid=(pl.cdiv(n, _BM),),
        in_specs=[
            pl.BlockSpec((_BM, n), lambda i: (i, 0)),
            pl.BlockSpec((n, din), lambda i: (0, 0)),
            pl.BlockSpec((din, dout), lambda i: (0, 0)),
            pl.BlockSpec((1, dout), lambda i: (0, 0)),
        ],
        out_specs=pl.BlockSpec((_BM, dout), lambda i: (i, 0)),
        out_shape=jax.ShapeDtypeStruct((n, dout), jnp.float32),
        compiler_params=pltpu.CompilerParams(
            dimension_semantics=("parallel",),
        ),
    )(adj, x, W, b2)
